# Initial kernel scaffold; baseline (speedup 1.0000x reference)
import jax
import jax.numpy as jnp
from jax.experimental import pallas as pl


def kernel(features, edge_index, W_w, W_b, attn_w, attn_b):
    raise NotImplementedError("write your pallas kernel here")


# speedup vs baseline: 13.8092x; 13.8092x over previous
import jax
import jax.numpy as jnp
from jax import lax
from jax.experimental import pallas as pl
from jax.experimental.pallas import tpu as pltpu
from jax.experimental.pallas import tpu_sc as plsc

N = 10000
NP = 10240
E = 320000
D = 128
DW = 8
NW = 32
EPW = E // NW
C = 80
NCHUNK = EPW // C
G = 25
NGROUP = NCHUNK // G
STRIPE = NP // 16


def _tc_prep(f_ref, ww_ref, wb_ref, aw_ref, ab_ref, h_ref, as_ref, ad_ref, m_ref):
    h = lax.dot_general(f_ref[...], ww_ref[...], (((1,), (1,)), ((), ())),
                        preferred_element_type=jnp.float32)
    h = h + wb_ref[...]
    h_ref[...] = h
    a_s = lax.dot_general(aw_ref[0:1], h, (((1,), (1,)), ((), ())),
                          preferred_element_type=jnp.float32) + ab_ref[0, 0]
    a_d = lax.dot_general(aw_ref[1:2], h, (((1,), (1,)), ((), ())),
                          preferred_element_type=jnp.float32)
    as_ref[...] = a_s
    ad_ref[...] = a_d
    pre_max = jnp.max(a_s) + jnp.max(a_d)
    m = jnp.where(pre_max > 0, pre_max, 0.2 * pre_max)
    m_ref[...] = jnp.full((1, 128), m, jnp.float32)


def _sc_edges(h_hbm, as_hbm, ad_hbm, m_hbm, src_hbm, dst_hbm, acc_hbm, den_hbm,
              src_g, dst_g, asv, adv, m_v, rows_v, dstage_v, ex_v,
              acc_sh, den_sh, sem):
    c = lax.axis_index("c")
    s = lax.axis_index("s")
    w = s * 2 + c

    pltpu.sync_copy(m_hbm.at[pl.ds(0, 16)], m_v)

    zero16 = jnp.zeros((16,), jnp.float32)
    zeros_i = jnp.zeros((16,), jnp.int32)
    lane = lax.iota(jnp.int32, 16)
    lane_div8 = lane >> 3
    lane_mod8 = lane & 7

    def zrow(r, carry):
        for d in range(D // 16):
            rows_v[r, pl.ds(d * 16, 16)] = zero16
        return carry

    lax.fori_loop(0, C, zrow, 0)

    def zden(r, carry):
        plsc.store_scatter(dstage_v, [r * 2 + lane_div8, lane_mod8], zero16)
        return carry

    lax.fori_loop(0, C // 2, zden, 0)

    base = s * STRIPE
    for k in range(STRIPE // C):
        pltpu.sync_copy(rows_v, acc_sh.at[pl.ds(base + k * C, C)])
        pltpu.sync_copy(dstage_v, den_sh.at[pl.ds(base + k * C, C)])
    plsc.subcore_barrier()

    mvec = m_v[...]

    def group_body(g, carry):
        pltpu.sync_copy(src_hbm.at[w, pl.ds(g * G, G)], src_g)
        pltpu.sync_copy(dst_hbm.at[w, pl.ds(g * G, G)], dst_g)

        def chunk_body(j, carry2):
            pltpu.sync_copy(h_hbm.at[src_g.at[j]], rows_v)
            pltpu.sync_copy(as_hbm.at[src_g.at[j]], asv)
            pltpu.sync_copy(ad_hbm.at[dst_g.at[j]], adv)

            def exblk(k, carry3):
                sl = pl.ds(k * 16, 16)
                pre = asv[sl] + adv[sl]
                sc = jnp.where(pre > 0, pre, 0.2 * pre)
                ex = jnp.exp(sc - mvec)
                ex_v[sl] = ex
                plsc.store_scatter(dstage_v, [lane + k * 16, zeros_i], ex)
                return carry3

            lax.fori_loop(0, C // 16, exblk, 0)

            def scale(r, carry3):
                exs = plsc.load_gather(ex_v, [zeros_i + r])
                for d in range(D // 16):
                    sl = pl.ds(d * 16, 16)
                    rows_v[r, sl] = rows_v[r, sl] * exs
                return carry3

            lax.fori_loop(0, C, scale, 0)

            cp1 = pltpu.async_copy(rows_v, acc_sh.at[dst_g.at[j]], sem, add=True)
            cp2 = pltpu.async_copy(dstage_v, den_sh.at[dst_g.at[j]], sem, add=True)
            cp1.wait()
            cp2.wait()
            return carry2

        lax.fori_loop(0, G, chunk_body, 0)
        return carry

    lax.fori_loop(0, NGROUP, group_body, 0)

    plsc.subcore_barrier()
    pltpu.sync_copy(acc_sh.at[pl.ds(base, STRIPE)],
                    acc_hbm.at[c, pl.ds(base, STRIPE)])
    pltpu.sync_copy(den_sh.at[pl.ds(base, STRIPE)],
                    den_hbm.at[c, pl.ds(base, STRIPE)])


def _tc_combine(acc_ref, den_ref, out_ref):
    num = acc_ref[0, :N] + acc_ref[1, :N]
    den = den_ref[0, :N, 0:1] + den_ref[1, :N, 0:1]
    out_ref[...] = jnp.where(den != 0.0, num / den, 0.0)


def kernel(features, edge_index, W_w, W_b, attn_w, attn_b):
    aw = attn_w.reshape(2, D)
    wb = W_b.reshape(1, D)
    ab = attn_b.reshape(1, 1)
    src = edge_index[0].reshape(NW, NCHUNK, C)
    dst = edge_index[1].reshape(NW, NCHUNK, C)

    h, as_row, ad_row, m_row = pl.pallas_call(
        _tc_prep,
        out_shape=[
            jax.ShapeDtypeStruct((N, D), jnp.float32),
            jax.ShapeDtypeStruct((1, N), jnp.float32),
            jax.ShapeDtypeStruct((1, N), jnp.float32),
            jax.ShapeDtypeStruct((1, 128), jnp.float32),
        ],
    )(features, W_w, wb, aw, ab)

    a_s = as_row.reshape(N)
    a_d = ad_row.reshape(N)
    m16 = m_row.reshape(128)[:16]

    sc_call = pl.kernel(
        _sc_edges,
        out_type=[
            jax.ShapeDtypeStruct((2, NP, D), jnp.float32),
            jax.ShapeDtypeStruct((2, NP, DW), jnp.float32),
        ],
        mesh=plsc.VectorSubcoreMesh(core_axis_name="c", subcore_axis_name="s"),
        compiler_params=pltpu.CompilerParams(
            use_tc_tiling_on_sc=False, needs_layout_passes=False),
        scratch_types=[
            pltpu.VMEM((G, C), jnp.int32),
            pltpu.VMEM((G, C), jnp.int32),
            pltpu.VMEM((C,), jnp.float32),
            pltpu.VMEM((C,), jnp.float32),
            pltpu.VMEM((16,), jnp.float32),
            pltpu.VMEM((C, D), jnp.float32),
            pltpu.VMEM((C, DW), jnp.float32),
            pltpu.VMEM((C,), jnp.float32),
            pltpu.VMEM_SHARED((NP, D), jnp.float32),
            pltpu.VMEM_SHARED((NP, DW), jnp.float32),
            pltpu.SemaphoreType.DMA,
        ],
    )
    acc, den = sc_call(h, a_s, a_d, m16, src, dst)

    out = pl.pallas_call(
        _tc_combine,
        out_shape=jax.ShapeDtypeStruct((N, D), jnp.float32),
    )(acc, den)
    return out


# speedup vs baseline: 34.6016x; 2.5057x over previous
import jax
import jax.numpy as jnp
from jax import lax
from jax.experimental import pallas as pl
from jax.experimental.pallas import tpu as pltpu
from jax.experimental.pallas import tpu_sc as plsc

N = 10000
NP = 10240
E = 320000
D = 128
DW = 8
NW = 32
EPW = E // NW
C = 80
NCHUNK = EPW // C
NBUF = 3
G = 25
NGROUP = NCHUNK // G
STRIPE = NP // 16


def _tc_prep(f_ref, ww_ref, wb_ref, aw_ref, ab_ref, h_ref, as_ref, ad_ref, m_ref):
    h = lax.dot_general(f_ref[...], ww_ref[...], (((1,), (1,)), ((), ())),
                        preferred_element_type=jnp.float32)
    h = h + wb_ref[...]
    h_ref[...] = h
    a_s = lax.dot_general(aw_ref[0:1], h, (((1,), (1,)), ((), ())),
                          preferred_element_type=jnp.float32) + ab_ref[0, 0]
    a_d = lax.dot_general(aw_ref[1:2], h, (((1,), (1,)), ((), ())),
                          preferred_element_type=jnp.float32)
    as_ref[...] = a_s
    ad_ref[...] = a_d
    pre_max = jnp.max(a_s) + jnp.max(a_d)
    m = jnp.where(pre_max > 0, pre_max, 0.2 * pre_max)
    m_ref[...] = jnp.full((1, 128), m, jnp.float32)


def _sc_edges(h_hbm, as_hbm, ad_hbm, m_hbm, src_hbm, dst_hbm, acc_hbm, den_hbm,
              src_g, dst_g, asv, adv, m_v, rows_v, dstage_v, ex_v,
              acc_sh, den_sh, gsem0, gsem1, gsem2, ssem0, ssem1, ssem2,
              isem0, isem1):
    c = lax.axis_index("c")
    s = lax.axis_index("s")
    w = s * 2 + c
    gsem = (gsem0, gsem1, gsem2)
    ssem = (ssem0, ssem1, ssem2)

    pltpu.sync_copy(m_hbm.at[pl.ds(0, 16)], m_v)
    pltpu.sync_copy(src_hbm.at[pl.ds(w * NCHUNK, G)], src_g.at[0])
    pltpu.sync_copy(dst_hbm.at[pl.ds(w * NCHUNK, G)], dst_g.at[0])

    def issue_group(gn):
        sl = lax.rem(gn, 2)
        start = w * NCHUNK + gn * G
        pltpu.async_copy(src_hbm.at[pl.ds(start, G)], src_g.at[sl], isem0)
        pltpu.async_copy(dst_hbm.at[pl.ds(start, G)], dst_g.at[sl], isem1)

    def wait_group(gn):
        sl = lax.rem(gn, 2)
        start = w * NCHUNK + gn * G
        pltpu.make_async_copy(src_hbm.at[pl.ds(start, G)], src_g.at[sl],
                              isem0).wait()
        pltpu.make_async_copy(dst_hbm.at[pl.ds(start, G)], dst_g.at[sl],
                              isem1).wait()

    zero16 = jnp.zeros((16,), jnp.float32)
    zeros_i = jnp.zeros((16,), jnp.int32)
    lane = lax.iota(jnp.int32, 16)
    lane_div8 = lane >> 3
    lane_mod8 = lane & 7

    def zrow(r, carry):
        for d in range(D // 16):
            rows_v[0, r, pl.ds(d * 16, 16)] = zero16
        return carry

    lax.fori_loop(0, C, zrow, 0)

    def zden(r, carry):
        for b in range(NBUF):
            plsc.store_scatter(dstage_v, [zeros_i + b, r * 2 + lane_div8,
                                          lane_mod8], zero16)
        return carry

    lax.fori_loop(0, C // 2, zden, 0)

    base = s * STRIPE
    for k in range(STRIPE // C):
        pltpu.sync_copy(rows_v.at[0], acc_sh.at[pl.ds(base + k * C, C)])
        pltpu.sync_copy(dstage_v.at[0], den_sh.at[pl.ds(base + k * C, C)])
    plsc.subcore_barrier()

    mvec = m_v[...]

    def idx_ref(g_ref, j):
        return g_ref.at[lax.rem(lax.div(j, G), 2), lax.rem(j, G)]

    def issue_gather(j, b):
        pltpu.async_copy(h_hbm.at[idx_ref(src_g, j)], rows_v.at[b], gsem[b])
        pltpu.async_copy(as_hbm.at[idx_ref(src_g, j)], asv.at[b], gsem[b])
        pltpu.async_copy(ad_hbm.at[idx_ref(dst_g, j)], adv.at[b], gsem[b])

    def wait_gather(j, b):
        pltpu.make_async_copy(h_hbm.at[idx_ref(src_g, j)], rows_v.at[b],
                              gsem[b]).wait()
        pltpu.make_async_copy(as_hbm.at[idx_ref(src_g, j)], asv.at[b],
                              gsem[b]).wait()
        pltpu.make_async_copy(ad_hbm.at[idx_ref(dst_g, j)], adv.at[b],
                              gsem[b]).wait()

    def issue_scatter(j, b):
        pltpu.async_copy(rows_v.at[b], acc_sh.at[idx_ref(dst_g, j)],
                         ssem[b], add=True)
        pltpu.async_copy(dstage_v.at[b], den_sh.at[idx_ref(dst_g, j)],
                         ssem[b], add=True)

    def wait_scatter(b):
        pltpu.make_async_copy(rows_v.at[b], acc_sh.at[pl.ds(0, C)], ssem[b]).wait()
        pltpu.make_async_copy(dstage_v.at[b], den_sh.at[pl.ds(0, C)], ssem[b]).wait()

    def compute(j, b):
        def exblk(k, carry):
            sl = pl.ds(k * 16, 16)
            pre = asv[b, sl] + adv[b, sl]
            sc = jnp.where(pre > 0, pre, 0.2 * pre)
            ex = jnp.exp(sc - mvec)
            ex_v[sl] = ex
            plsc.store_scatter(dstage_v, [zeros_i + b, lane + k * 16, zeros_i], ex)
            return carry

        lax.fori_loop(0, C // 16, exblk, 0)

        def scale(r, carry):
            exs = plsc.load_gather(ex_v, [zeros_i + r])
            for d in range(D // 16):
                sl = pl.ds(d * 16, 16)
                rows_v[b, r, sl] = rows_v[b, r, sl] * exs
            return carry

        lax.fori_loop(0, C, scale, 0)

    issue_gather(0, 0)
    issue_gather(1, 1)

    def tbody(t, carry):
        for b in range(NBUF):
            j = t * NBUF + b
            b2 = (b + 2) % NBUF

            @pl.when(j < NCHUNK)
            def _():
                wait_gather(j, b)
                compute(j, b)
                issue_scatter(j, b)

                @pl.when(j >= 1)
                def _():
                    wait_scatter(b2)

                @pl.when((lax.rem(j, G) == 0) & (j + G < NCHUNK))
                def _():
                    issue_group(lax.div(j, G) + 1)

                @pl.when(j + 2 < NCHUNK)
                def _():
                    @pl.when((lax.rem(j + 2, G) == 0) & (j + 2 > 0))
                    def _():
                        wait_group(lax.div(j + 2, G))

                    issue_gather(j + 2, b2)

        return carry

    lax.fori_loop(0, (NCHUNK + NBUF - 1) // NBUF, tbody, 0)
    wait_scatter((NCHUNK - 1) % NBUF)

    plsc.subcore_barrier()
    pltpu.sync_copy(acc_sh.at[pl.ds(base, STRIPE)],
                    acc_hbm.at[c, pl.ds(base, STRIPE)])
    pltpu.sync_copy(den_sh.at[pl.ds(base, STRIPE)],
                    den_hbm.at[c, pl.ds(base, STRIPE)])


def _tc_combine(acc_ref, den_ref, out_ref):
    num = acc_ref[0, :N] + acc_ref[1, :N]
    den = den_ref[0, :N, 0:1] + den_ref[1, :N, 0:1]
    out_ref[...] = jnp.where(den != 0.0, num / den, 0.0)


def kernel(features, edge_index, W_w, W_b, attn_w, attn_b):
    aw = attn_w.reshape(2, D)
    wb = W_b.reshape(1, D)
    ab = attn_b.reshape(1, 1)
    src = edge_index[0].reshape(NW * NCHUNK, C)
    dst = edge_index[1].reshape(NW * NCHUNK, C)

    h, as_row, ad_row, m_row = pl.pallas_call(
        _tc_prep,
        out_shape=[
            jax.ShapeDtypeStruct((N, D), jnp.float32),
            jax.ShapeDtypeStruct((1, N), jnp.float32),
            jax.ShapeDtypeStruct((1, N), jnp.float32),
            jax.ShapeDtypeStruct((1, 128), jnp.float32),
        ],
    )(features, W_w, wb, aw, ab)

    a_s = as_row.reshape(N)
    a_d = ad_row.reshape(N)
    m16 = m_row.reshape(128)[:16]

    sc_call = pl.kernel(
        _sc_edges,
        out_type=[
            jax.ShapeDtypeStruct((2, NP, D), jnp.float32),
            jax.ShapeDtypeStruct((2, NP, DW), jnp.float32),
        ],
        mesh=plsc.VectorSubcoreMesh(core_axis_name="c", subcore_axis_name="s"),
        compiler_params=pltpu.CompilerParams(
            use_tc_tiling_on_sc=False, needs_layout_passes=False),
        scratch_types=[
            pltpu.VMEM((2, G, C), jnp.int32),
            pltpu.VMEM((2, G, C), jnp.int32),
            pltpu.VMEM((NBUF, C), jnp.float32),
            pltpu.VMEM((NBUF, C), jnp.float32),
            pltpu.VMEM((16,), jnp.float32),
            pltpu.VMEM((NBUF, C, D), jnp.float32),
            pltpu.VMEM((NBUF, C, DW), jnp.float32),
            pltpu.VMEM((C,), jnp.float32),
            pltpu.VMEM_SHARED((NP, D), jnp.float32),
            pltpu.VMEM_SHARED((NP, DW), jnp.float32),
            pltpu.SemaphoreType.DMA,
            pltpu.SemaphoreType.DMA,
            pltpu.SemaphoreType.DMA,
            pltpu.SemaphoreType.DMA,
            pltpu.SemaphoreType.DMA,
            pltpu.SemaphoreType.DMA,
            pltpu.SemaphoreType.DMA,
            pltpu.SemaphoreType.DMA,
        ],
    )
    acc, den = sc_call(h, a_s, a_d, m16, src, dst)

    out = pl.pallas_call(
        _tc_combine,
        out_shape=jax.ShapeDtypeStruct((N, D), jnp.float32),
    )(acc, den)
    return out


# speedup vs baseline: 35.2810x; 1.0196x over previous
import jax
import jax.numpy as jnp
from jax import lax
from jax.experimental import pallas as pl
from jax.experimental.pallas import tpu as pltpu
from jax.experimental.pallas import tpu_sc as plsc

N = 10000
NP = 10240
E = 320000
D = 128
DW = 8
NW = 32
EPW = E // NW
C = 80
NCHUNK = EPW // C
NBUF = 3
G = 25
NGROUP = NCHUNK // G
STRIPE = NP // 16


def _tc_prep(f_ref, ww_ref, wb_ref, aw_ref, ab_ref, h_ref, as_ref, ad_ref, m_ref):
    h = lax.dot_general(f_ref[...], ww_ref[...], (((1,), (1,)), ((), ())),
                        preferred_element_type=jnp.float32)
    h = h + wb_ref[...]
    h_ref[...] = h
    a_s = lax.dot_general(aw_ref[0:1], h, (((1,), (1,)), ((), ())),
                          preferred_element_type=jnp.float32) + ab_ref[0, 0]
    a_d = lax.dot_general(aw_ref[1:2], h, (((1,), (1,)), ((), ())),
                          preferred_element_type=jnp.float32)
    as_ref[...] = a_s
    ad_ref[...] = a_d
    pre_max = jnp.max(a_s) + jnp.max(a_d)
    m = jnp.where(pre_max > 0, pre_max, 0.2 * pre_max)
    m_ref[...] = jnp.full((1, 128), m, jnp.float32)


def _sc_edges(h_hbm, as_hbm, ad_hbm, m_hbm, src_hbm, dst_hbm, acc_hbm, den_hbm,
              src_g, dst_g, asv, adv, m_v, rows_v, dstage_v, ex_v,
              acc_sh, den_sh, gsem0, gsem1, gsem2, ssem0, ssem1, ssem2,
              isem0, isem1):
    c = lax.axis_index("c")
    s = lax.axis_index("s")
    w = s * 2 + c
    gsem = (gsem0, gsem1, gsem2)
    ssem = (ssem0, ssem1, ssem2)

    pltpu.sync_copy(m_hbm.at[pl.ds(0, 16)], m_v)
    pltpu.sync_copy(src_hbm.at[pl.ds(w * NCHUNK, G)], src_g.at[0])
    pltpu.sync_copy(dst_hbm.at[pl.ds(w * NCHUNK, G)], dst_g.at[0])

    def issue_group(gn):
        sl = lax.rem(gn, 2)
        start = w * NCHUNK + gn * G
        pltpu.async_copy(src_hbm.at[pl.ds(start, G)], src_g.at[sl], isem0)
        pltpu.async_copy(dst_hbm.at[pl.ds(start, G)], dst_g.at[sl], isem1)

    def wait_group(gn):
        sl = lax.rem(gn, 2)
        start = w * NCHUNK + gn * G
        pltpu.make_async_copy(src_hbm.at[pl.ds(start, G)], src_g.at[sl],
                              isem0).wait()
        pltpu.make_async_copy(dst_hbm.at[pl.ds(start, G)], dst_g.at[sl],
                              isem1).wait()

    zero16 = jnp.zeros((16,), jnp.float32)
    zeros_i = jnp.zeros((16,), jnp.int32)
    lane = lax.iota(jnp.int32, 16)
    lane_div8 = lane >> 3
    lane_mod8 = lane & 7

    def zrow(r, carry):
        for d in range(D // 16):
            rows_v[0, r, pl.ds(d * 16, 16)] = zero16
        return carry

    lax.fori_loop(0, C, zrow, 0)

    def zden(r, carry):
        for b in range(NBUF):
            plsc.store_scatter(dstage_v, [zeros_i + b, r * 2 + lane_div8,
                                          lane_mod8], zero16)
        return carry

    lax.fori_loop(0, C // 2, zden, 0)

    base = s * STRIPE
    for k in range(STRIPE // C):
        pltpu.async_copy(rows_v.at[0], acc_sh.at[pl.ds(base + k * C, C)], ssem0)
        pltpu.async_copy(dstage_v.at[0], den_sh.at[pl.ds(base + k * C, C)], ssem1)
    for k in range(STRIPE // C):
        pltpu.make_async_copy(rows_v.at[0], acc_sh.at[pl.ds(base + k * C, C)],
                              ssem0).wait()
        pltpu.make_async_copy(dstage_v.at[0], den_sh.at[pl.ds(base + k * C, C)],
                              ssem1).wait()
    plsc.subcore_barrier()

    mvec = m_v[...]

    def idx_ref(g_ref, j):
        return g_ref.at[lax.rem(lax.div(j, G), 2), lax.rem(j, G)]

    def issue_gather(j, b):
        pltpu.async_copy(h_hbm.at[idx_ref(src_g, j)], rows_v.at[b], gsem[b])
        pltpu.async_copy(as_hbm.at[idx_ref(src_g, j)], asv.at[b], gsem[b])
        pltpu.async_copy(ad_hbm.at[idx_ref(dst_g, j)], adv.at[b], gsem[b])

    def wait_gather(j, b):
        pltpu.make_async_copy(h_hbm.at[idx_ref(src_g, j)], rows_v.at[b],
                              gsem[b]).wait()
        pltpu.make_async_copy(as_hbm.at[idx_ref(src_g, j)], asv.at[b],
                              gsem[b]).wait()
        pltpu.make_async_copy(ad_hbm.at[idx_ref(dst_g, j)], adv.at[b],
                              gsem[b]).wait()

    def issue_scatter(j, b):
        pltpu.async_copy(rows_v.at[b], acc_sh.at[idx_ref(dst_g, j)],
                         ssem[b], add=True)
        pltpu.async_copy(dstage_v.at[b], den_sh.at[idx_ref(dst_g, j)],
                         ssem[b], add=True)

    def wait_scatter(b):
        pltpu.make_async_copy(rows_v.at[b], acc_sh.at[pl.ds(0, C)], ssem[b]).wait()
        pltpu.make_async_copy(dstage_v.at[b], den_sh.at[pl.ds(0, C)], ssem[b]).wait()

    def compute(j, b):
        def exblk(k, carry):
            sl = pl.ds(k * 16, 16)
            pre = asv[b, sl] + adv[b, sl]
            sc = jnp.where(pre > 0, pre, 0.2 * pre)
            ex = jnp.exp(sc - mvec)
            ex_v[sl] = ex
            plsc.store_scatter(dstage_v, [zeros_i + b, lane + k * 16, zeros_i], ex)
            return carry

        lax.fori_loop(0, C // 16, exblk, 0)

        def scale(r2, carry):
            for u in range(2):
                r = r2 * 2 + u
                exs = plsc.load_gather(ex_v, [zeros_i + r])
                for d in range(D // 16):
                    sl = pl.ds(d * 16, 16)
                    rows_v[b, r, sl] = rows_v[b, r, sl] * exs
            return carry

        lax.fori_loop(0, C // 2, scale, 0)

    issue_gather(0, 0)
    issue_gather(1, 1)

    def tbody(t, carry):
        for b in range(NBUF):
            j = t * NBUF + b
            b2 = (b + 2) % NBUF

            @pl.when(j < NCHUNK)
            def _():
                wait_gather(j, b)
                compute(j, b)
                issue_scatter(j, b)

                @pl.when(j >= 1)
                def _():
                    wait_scatter(b2)

                @pl.when((lax.rem(j, G) == 0) & (j + G < NCHUNK))
                def _():
                    issue_group(lax.div(j, G) + 1)

                @pl.when(j + 2 < NCHUNK)
                def _():
                    @pl.when((lax.rem(j + 2, G) == 0) & (j + 2 > 0))
                    def _():
                        wait_group(lax.div(j + 2, G))

                    issue_gather(j + 2, b2)

        return carry

    lax.fori_loop(0, (NCHUNK + NBUF - 1) // NBUF, tbody, 0)
    wait_scatter((NCHUNK - 1) % NBUF)

    plsc.subcore_barrier()
    pltpu.async_copy(acc_sh.at[pl.ds(base, STRIPE)],
                     acc_hbm.at[c, pl.ds(base, STRIPE)], ssem0)
    pltpu.async_copy(den_sh.at[pl.ds(base, STRIPE)],
                     den_hbm.at[c, pl.ds(base, STRIPE)], ssem1)
    pltpu.make_async_copy(acc_sh.at[pl.ds(base, STRIPE)],
                          acc_hbm.at[c, pl.ds(base, STRIPE)], ssem0).wait()
    pltpu.make_async_copy(den_sh.at[pl.ds(base, STRIPE)],
                          den_hbm.at[c, pl.ds(base, STRIPE)], ssem1).wait()


def _tc_combine(acc_ref, den_ref, out_ref):
    num = acc_ref[0, :N] + acc_ref[1, :N]
    den = den_ref[0, :N, 0:1] + den_ref[1, :N, 0:1]
    out_ref[...] = jnp.where(den != 0.0, num / den, 0.0)


def kernel(features, edge_index, W_w, W_b, attn_w, attn_b):
    aw = attn_w.reshape(2, D)
    wb = W_b.reshape(1, D)
    ab = attn_b.reshape(1, 1)
    src = edge_index[0].reshape(NW * NCHUNK, C)
    dst = edge_index[1].reshape(NW * NCHUNK, C)

    h, as_row, ad_row, m_row = pl.pallas_call(
        _tc_prep,
        out_shape=[
            jax.ShapeDtypeStruct((N, D), jnp.float32),
            jax.ShapeDtypeStruct((1, N), jnp.float32),
            jax.ShapeDtypeStruct((1, N), jnp.float32),
            jax.ShapeDtypeStruct((1, 128), jnp.float32),
        ],
    )(features, W_w, wb, aw, ab)

    a_s = as_row.reshape(N)
    a_d = ad_row.reshape(N)
    m16 = m_row.reshape(128)[:16]

    sc_call = pl.kernel(
        _sc_edges,
        out_type=[
            jax.ShapeDtypeStruct((2, NP, D), jnp.float32),
            jax.ShapeDtypeStruct((2, NP, DW), jnp.float32),
        ],
        mesh=plsc.VectorSubcoreMesh(core_axis_name="c", subcore_axis_name="s"),
        compiler_params=pltpu.CompilerParams(
            use_tc_tiling_on_sc=False, needs_layout_passes=False),
        scratch_types=[
            pltpu.VMEM((2, G, C), jnp.int32),
            pltpu.VMEM((2, G, C), jnp.int32),
            pltpu.VMEM((NBUF, C), jnp.float32),
            pltpu.VMEM((NBUF, C), jnp.float32),
            pltpu.VMEM((16,), jnp.float32),
            pltpu.VMEM((NBUF, C, D), jnp.float32),
            pltpu.VMEM((NBUF, C, DW), jnp.float32),
            pltpu.VMEM((C,), jnp.float32),
            pltpu.VMEM_SHARED((NP, D), jnp.float32),
            pltpu.VMEM_SHARED((NP, DW), jnp.float32),
            pltpu.SemaphoreType.DMA,
            pltpu.SemaphoreType.DMA,
            pltpu.SemaphoreType.DMA,
            pltpu.SemaphoreType.DMA,
            pltpu.SemaphoreType.DMA,
            pltpu.SemaphoreType.DMA,
            pltpu.SemaphoreType.DMA,
            pltpu.SemaphoreType.DMA,
        ],
    )
    acc, den = sc_call(h, a_s, a_d, m16, src, dst)

    out = pl.pallas_call(
        _tc_combine,
        out_shape=jax.ShapeDtypeStruct((N, D), jnp.float32),
    )(acc, den)
    return out
